# Initial kernel scaffold; baseline (speedup 1.0000x reference)
import jax
import jax.numpy as jnp
from jax.experimental import pallas as pl


def kernel(x, params):
    raise NotImplementedError("write your pallas kernel here")


# speedup vs baseline: 212.7169x; 212.7169x over previous
import functools

import jax
import jax.numpy as jnp
from jax.experimental import pallas as pl
from jax.experimental.pallas import tpu as pltpu

_N_WIRES = 4
_DIM = 16
_N_LAYERS = 3
_BLK = 4096


def _variational_matrix_t(params):
    m = jnp.eye(_DIM, dtype=jnp.complex64)

    def apply_1q(s, u, q):
        left, right = 2 ** q, 2 ** (_N_WIRES - 1 - q)
        s = s.reshape(_DIM, left, 2, right)
        s = jnp.einsum('ij,bajc->baic', u, s,
                       precision=jax.lax.Precision.HIGHEST)
        return s.reshape(_DIM, _DIM)

    def apply_cnot(s, c):
        left = 2 ** c
        right = 2 ** (_N_WIRES - 2 - c)
        s = s.reshape(_DIM, left, 2, 2, right)
        s = s.at[:, :, 1].set(s[:, :, 1, ::-1])
        return s.reshape(_DIM, _DIM)

    for l in range(_N_LAYERS):
        for q in range(_N_WIRES):
            phi, th, om = params[l, q, 0], params[l, q, 1], params[l, q, 2]
            ephi = jnp.exp(-0.5j * phi.astype(jnp.complex64))
            zphi = jnp.zeros_like(ephi)
            rz_phi = jnp.stack([jnp.stack([ephi, zphi], -1),
                                jnp.stack([zphi, jnp.conj(ephi)], -1)], -2)
            cth = jnp.cos(th * 0.5)
            sth = jnp.sin(th * 0.5)
            ry_th = jnp.stack(
                [jnp.stack([cth, -sth], -1),
                 jnp.stack([sth, cth], -1)], -2).astype(jnp.complex64)
            eom = jnp.exp(-0.5j * om.astype(jnp.complex64))
            zom = jnp.zeros_like(eom)
            rz_om = jnp.stack([jnp.stack([eom, zom], -1),
                               jnp.stack([zom, jnp.conj(eom)], -1)], -2)
            u = jnp.matmul(rz_om,
                           jnp.matmul(ry_th, rz_phi,
                                      precision=jax.lax.Precision.HIGHEST),
                           precision=jax.lax.Precision.HIGHEST)
            m = apply_1q(m, u, q)
        for q in range(_N_WIRES - 1):
            m = apply_cnot(m, q)
    return m


def _qenc_kernel(xt_ref, vr_ref, vi_ref, o_ref):
    xt = xt_ref[...]
    half = xt * 0.5
    c = jnp.cos(half)
    s = jnp.sin(half)

    blk = xt.shape[1]
    sub = jax.lax.broadcasted_iota(jnp.int32, (_DIM, blk), 0)

    psi_r = None
    psi_i = None
    for i in range(_N_WIRES):
        ca = c[i:i + 1, :]
        sa = s[i:i + 1, :]
        cb = c[(i + 1) % 4:(i + 1) % 4 + 1, :]
        sb = s[(i + 1) % 4:(i + 1) % 4 + 1, :]
        cc = c[(i + 2) % 4:(i + 2) % 4 + 1, :]
        sc = s[(i + 2) % 4:(i + 2) % 4 + 1, :]
        cd = c[(i + 3) % 4:(i + 3) % 4 + 1, :]
        sd = s[(i + 3) % 4:(i + 3) % 4 + 1, :]
        u0r, u0i = ca * cb, -ca * sb
        u1r, u1i = sa * cb, sa * sb
        w0r = cc * u0r + sc * u1i
        w0i = cc * u0i - sc * u1r
        w1r = cc * u1r + sc * u0i
        w1i = cc * u1i - sc * u0r
        v0r = cd * w0r - sd * w1r
        v0i = cd * w0i - sd * w1i
        v1r = sd * w0r + cd * w1r
        v1i = sd * w0i + cd * w1i
        bit = jnp.bitwise_and(
            jax.lax.shift_right_logical(sub, 3 - i), 1) == 1
        vr = jnp.where(bit, jnp.broadcast_to(v1r, (_DIM, blk)),
                       jnp.broadcast_to(v0r, (_DIM, blk)))
        vi = jnp.where(bit, jnp.broadcast_to(v1i, (_DIM, blk)),
                       jnp.broadcast_to(v0i, (_DIM, blk)))
        if psi_r is None:
            psi_r, psi_i = vr, vi
        else:
            nr = psi_r * vr - psi_i * vi
            ni = psi_r * vi + psi_i * vr
            psi_r, psi_i = nr, ni

    vr_m = vr_ref[...]
    vi_m = vi_ref[...]
    dot = functools.partial(jnp.dot, preferred_element_type=jnp.float32,
                            precision=jax.lax.Precision.HIGHEST)
    amp_r = dot(vr_m, psi_r) - dot(vi_m, psi_i)
    amp_i = dot(vr_m, psi_i) + dot(vi_m, psi_r)
    probs = amp_r * amp_r + amp_i * amp_i

    row = jax.lax.broadcasted_iota(jnp.int32, (_N_WIRES, _DIM), 0)
    col = jax.lax.broadcasted_iota(jnp.int32, (_N_WIRES, _DIM), 1)
    sign = (1 - 2 * jnp.bitwise_and(
        jax.lax.shift_right_logical(col, 3 - row), 1)).astype(jnp.float32)
    o_ref[...] = dot(sign, probs)


@functools.partial(jax.jit, static_argnames=())
def kernel(x, params):
    b = x.shape[0]
    vt = _variational_matrix_t(params)
    v_r = jnp.real(vt).T
    v_i = jnp.imag(vt).T
    xt = x.T
    out_t = pl.pallas_call(
        _qenc_kernel,
        out_shape=jax.ShapeDtypeStruct((_N_WIRES, b), jnp.float32),
        grid=(b // _BLK,),
        in_specs=[
            pl.BlockSpec((_N_WIRES, _BLK), lambda i: (0, i)),
            pl.BlockSpec((_DIM, _DIM), lambda i: (0, 0)),
            pl.BlockSpec((_DIM, _DIM), lambda i: (0, 0)),
        ],
        out_specs=pl.BlockSpec((_N_WIRES, _BLK), lambda i: (0, i)),
        compiler_params=pltpu.CompilerParams(
            dimension_semantics=("parallel",),
        ),
        name="quantum_encoder",
    )(xt, v_r, v_i)
    return out_t.T
